# initial kernel scaffold (unmeasured)
import jax
import jax.numpy as jnp
from jax import lax
from jax.experimental import pallas as pl
from jax.experimental.pallas import tpu as pltpu

N_RING = 8
T = 4096
D = 2048
V_LOCAL = 8192
STRIPE = T // N_RING
HALF = STRIPE // 2


def _ring_pos(y, z):
    return jnp.where(y == 0, z, 2 * N_RING - 1 - N_RING - z + N_RING // 2 * 0) * 0 + jnp.where(y == 0, z, 7 - z)


def _ring_coords(p):
    y = jnp.where(p < 4, 0, 1)
    z = jnp.where(p < 4, p, 7 - p)
    return y, z


def kernel(ids, E):
    def body(x_ref, out_ref, p1_buf, p1_send, p1_recv,
             cw_send, cw_recv, ccw_send, ccw_recv):
        my_x = lax.axis_index("x")
        my_y = lax.axis_index("y")
        my_z = lax.axis_index("z")
        pos = _ring_pos(my_y, my_z)

        right = (pos + 1) % N_RING
        left = (pos - 1) % N_RING
        r_y, r_z = _ring_coords(right)
        l_y, l_z = _ring_coords(left)

        barrier = pltpu.get_barrier_semaphore()
        for dev in ((1 - my_x, my_y, my_z), (my_x, r_y, r_z), (my_x, l_y, l_z)):
            pl.semaphore_signal(
                barrier, inc=1, device_id=dev,
                device_id_type=pl.DeviceIdType.MESH,
            )
        pl.semaphore_wait(barrier, 3)

        p1 = pltpu.make_async_remote_copy(
            src_ref=x_ref,
            dst_ref=p1_buf,
            send_sem=p1_send,
            recv_sem=p1_recv,
            device_id=(1 - my_x, my_y, my_z),
            device_id_type=pl.DeviceIdType.MESH,
        )
        p1.start()
        p1.wait()
        out_ref[pl.ds(pos * STRIPE, STRIPE), :] = x_ref[:, :] + p1_buf[:, :]

        for h in range(N_RING - 1):
            o_cw = (pos - h) % N_RING
            o_ccw = (pos + h) % N_RING

            cw = pltpu.make_async_remote_copy(
                src_ref=out_ref.at[pl.ds(o_cw * STRIPE, HALF)],
                dst_ref=out_ref.at[pl.ds(o_cw * STRIPE, HALF)],
                send_sem=cw_send.at[h],
                recv_sem=cw_recv.at[h],
                device_id=(my_x, r_y, r_z),
                device_id_type=pl.DeviceIdType.MESH,
            )
            ccw = pltpu.make_async_remote_copy(
                src_ref=out_ref.at[pl.ds(o_ccw * STRIPE + HALF, HALF)],
                dst_ref=out_ref.at[pl.ds(o_ccw * STRIPE + HALF, HALF)],
                send_sem=ccw_send.at[h],
                recv_sem=ccw_recv.at[h],
                device_id=(my_x, l_y, l_z),
                device_id_type=pl.DeviceIdType.MESH,
            )
            cw.start()
            ccw.start()
            cw.wait()
            ccw.wait()

    my_x = lax.axis_index("x")
    my_y = lax.axis_index("y")
    my_z = lax.axis_index("z")
    pos = _ring_pos(my_y, my_z)
    ids_stripe = lax.dynamic_slice(ids, (pos * STRIPE,), (STRIPE,))
    local = ids_stripe - my_x * V_LOCAL
    valid = (local >= 0) & (local < V_LOCAL)
    rows = jnp.take(E, jnp.clip(local, 0, V_LOCAL - 1), axis=0)
    partial = jnp.where(valid[:, None], rows, 0.0).astype(jnp.float32)

    return pl.pallas_call(
        body,
        out_shape=jax.ShapeDtypeStruct((T, D), jnp.float32),
        in_specs=[pl.BlockSpec(memory_space=pltpu.VMEM)],
        out_specs=pl.BlockSpec(memory_space=pltpu.VMEM),
        scratch_shapes=[
            pltpu.VMEM((STRIPE, D), jnp.float32),
            pltpu.SemaphoreType.DMA,
            pltpu.SemaphoreType.DMA,
            pltpu.SemaphoreType.DMA((N_RING - 1,)),
            pltpu.SemaphoreType.DMA((N_RING - 1,)),
            pltpu.SemaphoreType.DMA((N_RING - 1,)),
            pltpu.SemaphoreType.DMA((N_RING - 1,)),
        ],
        compiler_params=pltpu.CompilerParams(collective_id=0),
    )(partial)


# baseline (device time: 280914 ns/iter reference)
import jax
import jax.numpy as jnp
from jax import lax
from jax.experimental import pallas as pl
from jax.experimental.pallas import tpu as pltpu

N_RING = 8
T = 4096
D = 2048
V_LOCAL = 8192
STRIPE = T // N_RING
HALF = STRIPE // 2


def _ring_pos(y, z):
    return jnp.where(y == 0, z, 7 - z)


def _ring_coords(p):
    y = jnp.where(p < 4, 0, 1)
    z = jnp.where(p < 4, p, 7 - p)
    return y, z


def kernel(ids, E):
    def body(x_ref, out_ref, p1_buf, p1_send, p1_recv,
             cw_send, cw_recv, ccw_send, ccw_recv):
        my_x = lax.axis_index("x")
        my_y = lax.axis_index("y")
        my_z = lax.axis_index("z")
        pos = _ring_pos(my_y, my_z)

        right = (pos + 1) % N_RING
        left = (pos - 1) % N_RING
        r_y, r_z = _ring_coords(right)
        l_y, l_z = _ring_coords(left)

        barrier = pltpu.get_barrier_semaphore()
        for dev in ((1 - my_x, my_y, my_z), (my_x, r_y, r_z), (my_x, l_y, l_z)):
            pl.semaphore_signal(
                barrier, inc=1, device_id=dev,
                device_id_type=pl.DeviceIdType.MESH,
            )
        pl.semaphore_wait(barrier, 3)

        p1 = pltpu.make_async_remote_copy(
            src_ref=x_ref,
            dst_ref=p1_buf,
            send_sem=p1_send,
            recv_sem=p1_recv,
            device_id=(1 - my_x, my_y, my_z),
            device_id_type=pl.DeviceIdType.MESH,
        )
        p1.start()
        p1.wait()
        out_ref[pl.ds(pos * STRIPE, STRIPE), :] = x_ref[:, :] + p1_buf[:, :]

        for h in range(N_RING - 1):
            o_cw = (pos - h) % N_RING
            o_ccw = (pos + h) % N_RING

            cw = pltpu.make_async_remote_copy(
                src_ref=out_ref.at[pl.ds(o_cw * STRIPE, HALF)],
                dst_ref=out_ref.at[pl.ds(o_cw * STRIPE, HALF)],
                send_sem=cw_send.at[h],
                recv_sem=cw_recv.at[h],
                device_id=(my_x, r_y, r_z),
                device_id_type=pl.DeviceIdType.MESH,
            )
            ccw = pltpu.make_async_remote_copy(
                src_ref=out_ref.at[pl.ds(o_ccw * STRIPE + HALF, HALF)],
                dst_ref=out_ref.at[pl.ds(o_ccw * STRIPE + HALF, HALF)],
                send_sem=ccw_send.at[h],
                recv_sem=ccw_recv.at[h],
                device_id=(my_x, l_y, l_z),
                device_id_type=pl.DeviceIdType.MESH,
            )
            cw.start()
            ccw.start()
            cw.wait()
            ccw.wait()

    my_x = lax.axis_index("x")
    my_y = lax.axis_index("y")
    my_z = lax.axis_index("z")
    pos = _ring_pos(my_y, my_z)
    ids_stripe = lax.dynamic_slice(ids, (pos * STRIPE,), (STRIPE,))
    local = ids_stripe - my_x * V_LOCAL
    valid = (local >= 0) & (local < V_LOCAL)
    rows = jnp.take(E, jnp.clip(local, 0, V_LOCAL - 1), axis=0)
    partial = jnp.where(valid[:, None], rows, 0.0).astype(jnp.float32)

    return pl.pallas_call(
        body,
        out_shape=jax.ShapeDtypeStruct((T, D), jnp.float32),
        in_specs=[pl.BlockSpec(memory_space=pltpu.VMEM)],
        out_specs=pl.BlockSpec(memory_space=pltpu.VMEM),
        scratch_shapes=[
            pltpu.VMEM((STRIPE, D), jnp.float32),
            pltpu.SemaphoreType.DMA,
            pltpu.SemaphoreType.DMA,
            pltpu.SemaphoreType.DMA((N_RING - 1,)),
            pltpu.SemaphoreType.DMA((N_RING - 1,)),
            pltpu.SemaphoreType.DMA((N_RING - 1,)),
            pltpu.SemaphoreType.DMA((N_RING - 1,)),
        ],
        compiler_params=pltpu.CompilerParams(
            collective_id=0,
            vmem_limit_bytes=60 * 1024 * 1024,
        ),
    )(partial)


# device time: 259203 ns/iter; 1.0838x vs baseline; 1.0838x over previous
import jax
import jax.numpy as jnp
from jax import lax
from jax.experimental import pallas as pl
from jax.experimental.pallas import tpu as pltpu

N_RING = 8
N_HOPS = N_RING - 1
T = 4096
D = 2048
V_LOCAL = 8192
STRIPE = T // N_RING
N_Q = 4
QROWS = STRIPE // N_Q
Q_ORDER = (0, 2, 1, 3)


def _ring_pos(y, z):
    return jnp.where(y == 0, z, 7 - z)


def _ring_coords(p):
    y = jnp.where(p < 4, 0, 1)
    z = jnp.where(p < 4, p, 7 - p)
    return y, z


def kernel(ids, E):
    def body(x_ref, out_ref, p1_buf, p1_send, p1_recv, ring_send, ring_recv):
        my_x = lax.axis_index("x")
        my_y = lax.axis_index("y")
        my_z = lax.axis_index("z")
        pos = _ring_pos(my_y, my_z)

        r_y, r_z = _ring_coords((pos + 1) % N_RING)
        l_y, l_z = _ring_coords((pos - 1) % N_RING)
        right_dev = (my_x, r_y, r_z)
        left_dev = (my_x, l_y, l_z)
        partner_dev = (1 - my_x, my_y, my_z)

        barrier = pltpu.get_barrier_semaphore()
        for dev in (partner_dev, right_dev, left_dev):
            pl.semaphore_signal(
                barrier, inc=1, device_id=dev,
                device_id_type=pl.DeviceIdType.MESH,
            )
        pl.semaphore_wait(barrier, 3)

        p1 = {}
        for q in Q_ORDER:
            p1[q] = pltpu.make_async_remote_copy(
                src_ref=x_ref.at[pl.ds(q * QROWS, QROWS)],
                dst_ref=p1_buf.at[pl.ds(q * QROWS, QROWS)],
                send_sem=p1_send.at[q],
                recv_sem=p1_recv.at[q],
                device_id=partner_dev,
                device_id_type=pl.DeviceIdType.MESH,
            )
            p1[q].start()

        ring = {}

        def start_hop(q, h):
            o = (pos - h) % N_RING if q < 2 else (pos + h) % N_RING
            rows = pl.ds(o * STRIPE + q * QROWS, QROWS)
            d = pltpu.make_async_remote_copy(
                src_ref=out_ref.at[rows],
                dst_ref=out_ref.at[rows],
                send_sem=ring_send.at[q, h],
                recv_sem=ring_recv.at[q, h],
                device_id=right_dev if q < 2 else left_dev,
                device_id_type=pl.DeviceIdType.MESH,
            )
            d.start()
            ring[(q, h)] = d

        for q in Q_ORDER:
            p1[q].wait_recv()
            qs = pl.ds(q * QROWS, QROWS)
            out_ref[pl.ds(pos * STRIPE + q * QROWS, QROWS), :] = (
                x_ref[qs, :] + p1_buf[qs, :]
            )
            start_hop(q, 0)

        for h in range(1, N_HOPS):
            for q in Q_ORDER:
                ring[(q, h - 1)].wait_recv()
                start_hop(q, h)
        for q in Q_ORDER:
            ring[(q, N_HOPS - 1)].wait_recv()

        for q in Q_ORDER:
            p1[q].wait_send()
        for d in ring.values():
            d.wait_send()

    my_x = lax.axis_index("x")
    my_y = lax.axis_index("y")
    my_z = lax.axis_index("z")
    pos = _ring_pos(my_y, my_z)
    ids_stripe = lax.dynamic_slice(ids, (pos * STRIPE,), (STRIPE,))
    local = ids_stripe - my_x * V_LOCAL
    valid = (local >= 0) & (local < V_LOCAL)
    rows = jnp.take(E, jnp.clip(local, 0, V_LOCAL - 1), axis=0)
    partial = jnp.where(valid[:, None], rows, 0.0).astype(jnp.float32)

    return pl.pallas_call(
        body,
        out_shape=jax.ShapeDtypeStruct((T, D), jnp.float32),
        in_specs=[pl.BlockSpec(memory_space=pltpu.VMEM)],
        out_specs=pl.BlockSpec(memory_space=pltpu.VMEM),
        scratch_shapes=[
            pltpu.VMEM((STRIPE, D), jnp.float32),
            pltpu.SemaphoreType.DMA((N_Q,)),
            pltpu.SemaphoreType.DMA((N_Q,)),
            pltpu.SemaphoreType.DMA((N_Q, N_HOPS)),
            pltpu.SemaphoreType.DMA((N_Q, N_HOPS)),
        ],
        compiler_params=pltpu.CompilerParams(
            collective_id=0,
            vmem_limit_bytes=60 * 1024 * 1024,
        ),
    )(partial)


# device time: 162643 ns/iter; 1.7272x vs baseline; 1.5937x over previous
import jax
import jax.numpy as jnp
from jax import lax
from jax.experimental import pallas as pl
from jax.experimental.pallas import tpu as pltpu

N_RING = 8
N_HOPS = N_RING - 1
T = 4096
D = 2048
V_LOCAL = 8192
STRIPE = T // N_RING
N_Q = 4
QROWS = STRIPE // N_Q
Q_ORDER = (0, 2, 1, 3)


def _ring_pos(y, z):
    return jnp.where(y == 0, z, 7 - z)


def _ring_coords(p):
    y = jnp.where(p < 4, 0, 1)
    z = jnp.where(p < 4, p, 7 - p)
    return y, z


def kernel(ids, E):
    def body(x_ref, out_ref, comm, p1_buf, p1_send, p1_recv,
             ring_send, ring_recv):
        my_x = lax.axis_index("x")
        my_y = lax.axis_index("y")
        my_z = lax.axis_index("z")
        pos = _ring_pos(my_y, my_z)

        r_y, r_z = _ring_coords((pos + 1) % N_RING)
        l_y, l_z = _ring_coords((pos - 1) % N_RING)
        right_dev = (my_x, r_y, r_z)
        left_dev = (my_x, l_y, l_z)
        partner_dev = (1 - my_x, my_y, my_z)

        barrier = pltpu.get_barrier_semaphore()
        for dev in (partner_dev, right_dev, left_dev):
            pl.semaphore_signal(
                barrier, inc=1, device_id=dev,
                device_id_type=pl.DeviceIdType.MESH,
            )
        pl.semaphore_wait(barrier, 3)

        p1 = {}
        for q in Q_ORDER:
            p1[q] = pltpu.make_async_remote_copy(
                src_ref=x_ref.at[pl.ds(q * QROWS, QROWS)],
                dst_ref=p1_buf.at[pl.ds(q * QROWS, QROWS)],
                send_sem=p1_send.at[q],
                recv_sem=p1_recv.at[q],
                device_id=partner_dev,
                device_id_type=pl.DeviceIdType.MESH,
            )
            p1[q].start()

        ring = {}

        def chunk_rows(q, h):
            o = (pos - h) % N_RING if q < 2 else (pos + h) % N_RING
            return pl.ds(o * STRIPE + q * QROWS, QROWS)

        def start_hop(q, h):
            rows = chunk_rows(q, h)
            d = pltpu.make_async_remote_copy(
                src_ref=comm.at[rows],
                dst_ref=comm.at[rows],
                send_sem=ring_send.at[q, h],
                recv_sem=ring_recv.at[q, h],
                device_id=right_dev if q < 2 else left_dev,
                device_id_type=pl.DeviceIdType.MESH,
            )
            d.start()
            ring[(q, h)] = d

        for q in Q_ORDER:
            p1[q].wait_recv()
            qs = pl.ds(q * QROWS, QROWS)
            rows = pl.ds(pos * STRIPE + q * QROWS, QROWS)
            comm[rows, :] = x_ref[qs, :] + p1_buf[qs, :]
            start_hop(q, 0)
            out_ref[rows, :] = comm[rows, :].astype(jnp.float32)

        for h in range(1, N_HOPS):
            for q in Q_ORDER:
                ring[(q, h - 1)].wait_recv()
                start_hop(q, h)
                rows = chunk_rows(q, h)
                out_ref[rows, :] = comm[rows, :].astype(jnp.float32)
        for q in Q_ORDER:
            ring[(q, N_HOPS - 1)].wait_recv()
            rows = chunk_rows(q, N_HOPS)
            out_ref[rows, :] = comm[rows, :].astype(jnp.float32)

        for q in Q_ORDER:
            p1[q].wait_send()
        for d in ring.values():
            d.wait_send()

    my_x = lax.axis_index("x")
    my_y = lax.axis_index("y")
    my_z = lax.axis_index("z")
    pos = _ring_pos(my_y, my_z)
    ids_stripe = lax.dynamic_slice(ids, (pos * STRIPE,), (STRIPE,))
    local = ids_stripe - my_x * V_LOCAL
    valid = (local >= 0) & (local < V_LOCAL)
    rows = jnp.take(E, jnp.clip(local, 0, V_LOCAL - 1), axis=0)
    partial = jnp.where(valid[:, None], rows, 0.0).astype(jnp.bfloat16)

    return pl.pallas_call(
        body,
        out_shape=jax.ShapeDtypeStruct((T, D), jnp.float32),
        in_specs=[pl.BlockSpec(memory_space=pltpu.VMEM)],
        out_specs=pl.BlockSpec(memory_space=pltpu.VMEM),
        scratch_shapes=[
            pltpu.VMEM((T, D), jnp.bfloat16),
            pltpu.VMEM((STRIPE, D), jnp.bfloat16),
            pltpu.SemaphoreType.DMA((N_Q,)),
            pltpu.SemaphoreType.DMA((N_Q,)),
            pltpu.SemaphoreType.DMA((N_Q, N_HOPS)),
            pltpu.SemaphoreType.DMA((N_Q, N_HOPS)),
        ],
        compiler_params=pltpu.CompilerParams(
            collective_id=0,
            vmem_limit_bytes=60 * 1024 * 1024,
        ),
    )(partial)


# device time: 162489 ns/iter; 1.7288x vs baseline; 1.0009x over previous
import jax
import jax.numpy as jnp
from jax import lax
from jax.experimental import pallas as pl
from jax.experimental.pallas import tpu as pltpu

N_RING = 8
N_HOPS = N_RING - 1
T = 4096
D = 2048
V_LOCAL = 8192
STRIPE = T // N_RING
N_Q = 4
QROWS = STRIPE // N_Q
Q_ORDER = (0, 2, 1, 3)


def _ring_pos(y, z):
    return jnp.where(y == 0, z, 7 - z)


def _ring_coords(p):
    y = jnp.where(p < 4, 0, 1)
    z = jnp.where(p < 4, p, 7 - p)
    return y, z


def kernel(ids, E):
    def body(x_ref, hbm_out, comm, out_ref, p1_buf, p1_send, p1_recv,
             ring_send, ring_recv, out_dma_sem):
        my_x = lax.axis_index("x")
        my_y = lax.axis_index("y")
        my_z = lax.axis_index("z")
        pos = _ring_pos(my_y, my_z)

        r_y, r_z = _ring_coords((pos + 1) % N_RING)
        l_y, l_z = _ring_coords((pos - 1) % N_RING)
        right_dev = (my_x, r_y, r_z)
        left_dev = (my_x, l_y, l_z)
        partner_dev = (1 - my_x, my_y, my_z)

        barrier = pltpu.get_barrier_semaphore()
        for dev in (partner_dev, right_dev, left_dev):
            pl.semaphore_signal(
                barrier, inc=1, device_id=dev,
                device_id_type=pl.DeviceIdType.MESH,
            )
        pl.semaphore_wait(barrier, 3)

        p1 = {}
        for q in Q_ORDER:
            p1[q] = pltpu.make_async_remote_copy(
                src_ref=x_ref.at[pl.ds(q * QROWS, QROWS)],
                dst_ref=p1_buf.at[pl.ds(q * QROWS, QROWS)],
                send_sem=p1_send.at[q],
                recv_sem=p1_recv.at[q],
                device_id=partner_dev,
                device_id_type=pl.DeviceIdType.MESH,
            )
            p1[q].start()

        ring = {}

        def chunk_rows(q, h):
            o = (pos - h) % N_RING if q < 2 else (pos + h) % N_RING
            return pl.ds(o * STRIPE + q * QROWS, QROWS)

        def start_hop(q, h):
            rows = chunk_rows(q, h)
            d = pltpu.make_async_remote_copy(
                src_ref=comm.at[rows],
                dst_ref=comm.at[rows],
                send_sem=ring_send.at[q, h],
                recv_sem=ring_recv.at[q, h],
                device_id=right_dev if q < 2 else left_dev,
                device_id_type=pl.DeviceIdType.MESH,
            )
            d.start()
            ring[(q, h)] = d

        for q in Q_ORDER:
            p1[q].wait_recv()
            qs = pl.ds(q * QROWS, QROWS)
            rows = pl.ds(pos * STRIPE + q * QROWS, QROWS)
            comm[rows, :] = x_ref[qs, :] + p1_buf[qs, :]
            start_hop(q, 0)
            out_ref[rows, :] = comm[rows, :].astype(jnp.float32)

        for h in range(1, N_HOPS):
            for q in Q_ORDER:
                ring[(q, h - 1)].wait_recv()
                start_hop(q, h)
                rows = chunk_rows(q, h)
                out_ref[rows, :] = comm[rows, :].astype(jnp.float32)
        for q in Q_ORDER:
            ring[(q, N_HOPS - 1)].wait_recv()
            rows = chunk_rows(q, N_HOPS)
            out_ref[rows, :] = comm[rows, :].astype(jnp.float32)

        out_copy = pltpu.make_async_copy(out_ref, hbm_out, out_dma_sem)
        out_copy.start()

        for q in Q_ORDER:
            p1[q].wait_send()
        for d in ring.values():
            d.wait_send()
        out_copy.wait()

    my_x = lax.axis_index("x")
    my_y = lax.axis_index("y")
    my_z = lax.axis_index("z")
    pos = _ring_pos(my_y, my_z)
    ids_stripe = lax.dynamic_slice(ids, (pos * STRIPE,), (STRIPE,))
    local = ids_stripe - my_x * V_LOCAL
    valid = (local >= 0) & (local < V_LOCAL)
    rows = jnp.take(E, jnp.clip(local, 0, V_LOCAL - 1), axis=0)
    partial = jnp.where(valid[:, None], rows, 0.0).astype(jnp.bfloat16)

    return pl.pallas_call(
        body,
        out_shape=jax.ShapeDtypeStruct((T, D), jnp.float32),
        in_specs=[pl.BlockSpec(memory_space=pltpu.VMEM)],
        out_specs=pl.BlockSpec(memory_space=pl.ANY),
        scratch_shapes=[
            pltpu.VMEM((T, D), jnp.bfloat16),
            pltpu.VMEM((T, D), jnp.float32),
            pltpu.VMEM((STRIPE, D), jnp.bfloat16),
            pltpu.SemaphoreType.DMA((N_Q,)),
            pltpu.SemaphoreType.DMA((N_Q,)),
            pltpu.SemaphoreType.DMA((N_Q, N_HOPS)),
            pltpu.SemaphoreType.DMA((N_Q, N_HOPS)),
            pltpu.SemaphoreType.DMA,
        ],
        compiler_params=pltpu.CompilerParams(
            collective_id=0,
            vmem_limit_bytes=60 * 1024 * 1024,
        ),
    )(partial)


# device time: 152898 ns/iter; 1.8373x vs baseline; 1.0627x over previous
import jax
import jax.numpy as jnp
from jax import lax
from jax.experimental import pallas as pl
from jax.experimental.pallas import tpu as pltpu

N_RING = 8
N_HOPS = N_RING - 1
T = 4096
D = 2048
V_LOCAL = 8192
STRIPE = T // N_RING
N_Q = 4
QROWS = STRIPE // N_Q
Q_ORDER = (0, 2, 1, 3)


def _ring_pos(y, z):
    return jnp.where(y == 0, z, 7 - z)


def _ring_coords(p):
    y = jnp.where(p < 4, 0, 1)
    z = jnp.where(p < 4, p, 7 - p)
    return y, z


def kernel(ids, E):
    def body(x_ref, hbm_out, comm, out_ref, p1_buf, p1_send, p1_recv,
             ring_send, ring_recv, out_dma_sems):
        my_x = lax.axis_index("x")
        my_y = lax.axis_index("y")
        my_z = lax.axis_index("z")
        pos = _ring_pos(my_y, my_z)

        r_y, r_z = _ring_coords((pos + 1) % N_RING)
        l_y, l_z = _ring_coords((pos - 1) % N_RING)
        right_dev = (my_x, r_y, r_z)
        left_dev = (my_x, l_y, l_z)
        partner_dev = (1 - my_x, my_y, my_z)

        barrier = pltpu.get_barrier_semaphore()
        for dev in (partner_dev, right_dev, left_dev):
            pl.semaphore_signal(
                barrier, inc=1, device_id=dev,
                device_id_type=pl.DeviceIdType.MESH,
            )
        pl.semaphore_wait(barrier, 3)

        p1 = {}
        for q in Q_ORDER:
            p1[q] = pltpu.make_async_remote_copy(
                src_ref=x_ref.at[pl.ds(q * QROWS, QROWS)],
                dst_ref=p1_buf.at[pl.ds(q * QROWS, QROWS)],
                send_sem=p1_send.at[q],
                recv_sem=p1_recv.at[q],
                device_id=partner_dev,
                device_id_type=pl.DeviceIdType.MESH,
            )
            p1[q].start()

        ring = {}
        out_dmas = []

        def cast_and_ship(rows, q, j):
            out_ref[rows, :] = comm[rows, :].astype(jnp.float32)
            d = pltpu.make_async_copy(
                out_ref.at[rows], hbm_out.at[rows], out_dma_sems.at[q, j]
            )
            d.start()
            out_dmas.append(d)

        def chunk_rows(q, h):
            o = (pos - h) % N_RING if q < 2 else (pos + h) % N_RING
            return pl.ds(o * STRIPE + q * QROWS, QROWS)

        def start_hop(q, h):
            rows = chunk_rows(q, h)
            d = pltpu.make_async_remote_copy(
                src_ref=comm.at[rows],
                dst_ref=comm.at[rows],
                send_sem=ring_send.at[q, h],
                recv_sem=ring_recv.at[q, h],
                device_id=right_dev if q < 2 else left_dev,
                device_id_type=pl.DeviceIdType.MESH,
            )
            d.start()
            ring[(q, h)] = d

        for q in Q_ORDER:
            p1[q].wait_recv()
            qs = pl.ds(q * QROWS, QROWS)
            rows = pl.ds(pos * STRIPE + q * QROWS, QROWS)
            comm[rows, :] = x_ref[qs, :] + p1_buf[qs, :]
            start_hop(q, 0)
            cast_and_ship(rows, q, 0)

        for h in range(1, N_HOPS):
            for q in Q_ORDER:
                ring[(q, h - 1)].wait_recv()
                start_hop(q, h)
                cast_and_ship(chunk_rows(q, h), q, h)
        for q in Q_ORDER:
            ring[(q, N_HOPS - 1)].wait_recv()
            cast_and_ship(chunk_rows(q, N_HOPS), q, N_HOPS)

        for q in Q_ORDER:
            p1[q].wait_send()
        for d in ring.values():
            d.wait_send()
        for d in out_dmas:
            d.wait()

    my_x = lax.axis_index("x")
    my_y = lax.axis_index("y")
    my_z = lax.axis_index("z")
    pos = _ring_pos(my_y, my_z)
    ids_stripe = lax.dynamic_slice(ids, (pos * STRIPE,), (STRIPE,))
    local = ids_stripe - my_x * V_LOCAL
    valid = (local >= 0) & (local < V_LOCAL)
    rows = jnp.take(E, jnp.clip(local, 0, V_LOCAL - 1), axis=0)
    partial = jnp.where(valid[:, None], rows, 0.0).astype(jnp.bfloat16)

    return pl.pallas_call(
        body,
        out_shape=jax.ShapeDtypeStruct((T, D), jnp.float32),
        in_specs=[pl.BlockSpec(memory_space=pltpu.VMEM)],
        out_specs=pl.BlockSpec(memory_space=pl.ANY),
        scratch_shapes=[
            pltpu.VMEM((T, D), jnp.bfloat16),
            pltpu.VMEM((T, D), jnp.float32),
            pltpu.VMEM((STRIPE, D), jnp.bfloat16),
            pltpu.SemaphoreType.DMA((N_Q,)),
            pltpu.SemaphoreType.DMA((N_Q,)),
            pltpu.SemaphoreType.DMA((N_Q, N_HOPS)),
            pltpu.SemaphoreType.DMA((N_Q, N_HOPS)),
            pltpu.SemaphoreType.DMA((N_Q, N_RING)),
        ],
        compiler_params=pltpu.CompilerParams(
            collective_id=0,
            vmem_limit_bytes=60 * 1024 * 1024,
        ),
    )(partial)


# device time: 135719 ns/iter; 2.0698x vs baseline; 1.1266x over previous
import jax
import jax.numpy as jnp
from jax import lax
from jax.experimental import pallas as pl
from jax.experimental.pallas import tpu as pltpu

N_RING = 8
N_HOPS = N_RING - 1
T = 4096
D = 2048
V_LOCAL = 8192
STRIPE = T // N_RING
ER = STRIPE // 8
L_ORDER = (0, 3, 1, 4, 2, 5)


def _ring_pos(y, z):
    return jnp.where(y == 0, z, 7 - z)


def _ring_coords(p):
    y = jnp.where(p < 4, 0, 1)
    z = jnp.where(p < 4, p, 7 - p)
    return y, z


def kernel(ids, E):
    def body(x_ref, hbm_out, comm, out_stage, p1_buf,
             p1_send, p1_recv, ring_send, ring_recv, mir_send, mir_recv,
             out_ring_sems, out_mir_sems, out_own_sem):
        p = lax.axis_index("x")
        my_y = lax.axis_index("y")
        my_z = lax.axis_index("z")
        pos = _ring_pos(my_y, my_z)
        mb = p * 2 * ER

        r_y, r_z = _ring_coords((pos + 1) % N_RING)
        l_y, l_z = _ring_coords((pos - 1) % N_RING)
        right_dev = (p, r_y, r_z)
        left_dev = (p, l_y, l_z)
        partner_dev = (1 - p, my_y, my_z)

        barrier = pltpu.get_barrier_semaphore()
        for dev in (partner_dev, right_dev, left_dev):
            pl.semaphore_signal(
                barrier, inc=1, device_id=dev,
                device_id_type=pl.DeviceIdType.MESH,
            )
        pl.semaphore_wait(barrier, 3)

        pb = (1 - p) * 2 * ER
        p1 = {}
        for L in L_ORDER:
            p1[L] = pltpu.make_async_remote_copy(
                src_ref=x_ref.at[pl.ds(pb + L * ER, ER)],
                dst_ref=p1_buf.at[pl.ds(L * ER, ER)],
                send_sem=p1_send.at[L],
                recv_sem=p1_recv.at[L],
                device_id=partner_dev,
                device_id_type=pl.DeviceIdType.MESH,
            )
            p1[L].start()

        ring = {}
        mirrors = []
        out_dmas = []

        def ship(rows, sem):
            out_stage[rows, :] = comm[rows, :].astype(jnp.float32)
            d = pltpu.make_async_copy(
                out_stage.at[rows], hbm_out.at[rows], sem
            )
            d.start()
            out_dmas.append(d)

        def start_hop(L, h):
            o = (pos - h) % N_RING if L < 3 else (pos + h) % N_RING
            rows = pl.ds(o * STRIPE + mb + L * ER, ER)
            d = pltpu.make_async_remote_copy(
                src_ref=comm.at[rows],
                dst_ref=comm.at[rows],
                send_sem=ring_send.at[L, h],
                recv_sem=ring_recv.at[L, h],
                device_id=right_dev if L < 3 else left_dev,
                device_id_type=pl.DeviceIdType.MESH,
            )
            d.start()
            ring[(L, h)] = d

        def mirror_send(s, h):
            o = jnp.where(p == 0, (pos - h) % N_RING, (pos + h) % N_RING)
            rows = pl.ds(o * STRIPE + p * 6 * ER, 2 * ER)
            d = pltpu.make_async_remote_copy(
                src_ref=comm.at[rows],
                dst_ref=comm.at[rows],
                send_sem=mir_send.at[s],
                recv_sem=mir_recv.at[s],
                device_id=partner_dev,
                device_id_type=pl.DeviceIdType.MESH,
            )
            d.start()
            mirrors.append(d)

        def mirror_wait_ship(s):
            o = jnp.where(p == 0, (pos + s) % N_RING, (pos - s) % N_RING)
            rows = pl.ds(o * STRIPE + (1 - p) * 6 * ER, 2 * ER)
            d = pltpu.make_async_remote_copy(
                src_ref=comm.at[rows],
                dst_ref=comm.at[rows],
                send_sem=mir_send.at[s],
                recv_sem=mir_recv.at[s],
                device_id=partner_dev,
                device_id_type=pl.DeviceIdType.MESH,
            )
            d.wait_recv()
            ship(rows, out_mir_sems.at[s])

        for L in L_ORDER:
            p1[L].wait_recv()
            rows = pl.ds(pos * STRIPE + mb + L * ER, ER)
            comm[rows, :] = (
                x_ref[pl.ds(mb + L * ER, ER), :]
                + p1_buf[pl.ds(L * ER, ER), :]
            )
            start_hop(L, 0)
        mirror_send(0, 0)
        ship(pl.ds(pos * STRIPE + mb, 6 * ER), out_own_sem)

        for h in range(1, N_HOPS + 1):
            last = h == N_HOPS
            for L in L_ORDER:
                ring[(L, h - 1)].wait_recv()
                if not last:
                    start_hop(L, h)
            mirror_send(h, h)
            ship(
                pl.ds(((pos - h) % N_RING) * STRIPE + mb, 3 * ER),
                out_ring_sems.at[0, h - 1],
            )
            ship(
                pl.ds(((pos + h) % N_RING) * STRIPE + mb + 3 * ER, 3 * ER),
                out_ring_sems.at[1, h - 1],
            )
            mirror_wait_ship(h - 1)
        mirror_wait_ship(N_HOPS)

        for L in L_ORDER:
            p1[L].wait_send()
        for d in ring.values():
            d.wait_send()
        for d in mirrors:
            d.wait_send()
        for d in out_dmas:
            d.wait()

    my_x = lax.axis_index("x")
    my_y = lax.axis_index("y")
    my_z = lax.axis_index("z")
    pos = _ring_pos(my_y, my_z)
    ids_stripe = lax.dynamic_slice(ids, (pos * STRIPE,), (STRIPE,))
    local = ids_stripe - my_x * V_LOCAL
    valid = (local >= 0) & (local < V_LOCAL)
    rows = jnp.take(E, jnp.clip(local, 0, V_LOCAL - 1), axis=0)
    partial = jnp.where(valid[:, None], rows, 0.0).astype(jnp.bfloat16)

    return pl.pallas_call(
        body,
        out_shape=jax.ShapeDtypeStruct((T, D), jnp.float32),
        in_specs=[pl.BlockSpec(memory_space=pltpu.VMEM)],
        out_specs=pl.BlockSpec(memory_space=pl.ANY),
        scratch_shapes=[
            pltpu.VMEM((T, D), jnp.bfloat16),
            pltpu.VMEM((T, D), jnp.float32),
            pltpu.VMEM((6 * ER, D), jnp.bfloat16),
            pltpu.SemaphoreType.DMA((6,)),
            pltpu.SemaphoreType.DMA((6,)),
            pltpu.SemaphoreType.DMA((6, N_HOPS)),
            pltpu.SemaphoreType.DMA((6, N_HOPS)),
            pltpu.SemaphoreType.DMA((N_RING,)),
            pltpu.SemaphoreType.DMA((N_RING,)),
            pltpu.SemaphoreType.DMA((2, N_HOPS)),
            pltpu.SemaphoreType.DMA((N_RING,)),
            pltpu.SemaphoreType.DMA,
        ],
        compiler_params=pltpu.CompilerParams(
            collective_id=0,
            vmem_limit_bytes=60 * 1024 * 1024,
        ),
    )(partial)


# device time: 132269 ns/iter; 2.1238x vs baseline; 1.0261x over previous
import jax
import jax.numpy as jnp
from jax import lax
from jax.experimental import pallas as pl
from jax.experimental.pallas import tpu as pltpu

N_RING = 8
N_HOPS = N_RING - 1
T = 4096
D = 2048
V_LOCAL = 8192
STRIPE = T // N_RING
ER = STRIPE // 8
L_ORDER = (0, 3, 1, 4, 2, 5)


def _ring_pos(y, z):
    return jnp.where(y == 0, z, 7 - z)


def _ring_coords(p):
    y = jnp.where(p < 4, 0, 1)
    z = jnp.where(p < 4, p, 7 - p)
    return y, z


def kernel(ids, E):
    def body(E_ref, idsc_ref, valid_ref, hbm_out, comm, out_stage, p1_buf,
             gbuf, xbuf, g_sems,
             p1_send, p1_recv, ring_send, ring_recv, mir_send, mir_recv,
             out_ring_sems, out_mir_sems, out_own_sem):
        p = lax.axis_index("x")
        my_y = lax.axis_index("y")
        my_z = lax.axis_index("z")
        pos = _ring_pos(my_y, my_z)
        mb = p * 2 * ER

        r_y, r_z = _ring_coords((pos + 1) % N_RING)
        l_y, l_z = _ring_coords((pos - 1) % N_RING)
        right_dev = (p, r_y, r_z)
        left_dev = (p, l_y, l_z)
        partner_dev = (1 - p, my_y, my_z)

        for e in range(8):
            base = e * ER

            def issue(i, c, base=base, e=e):
                r = base + i
                pltpu.make_async_copy(
                    E_ref.at[pl.ds(idsc_ref[r], 1)],
                    gbuf.at[pl.ds(r, 1)],
                    g_sems.at[e],
                ).start()
                return c

            lax.fori_loop(0, ER, issue, 0, unroll=8)

        barrier = pltpu.get_barrier_semaphore()
        for dev in (partner_dev, right_dev, left_dev):
            pl.semaphore_signal(
                barrier, inc=1, device_id=dev,
                device_id_type=pl.DeviceIdType.MESH,
            )
        pl.semaphore_wait(barrier, 3)

        for e in range(8):
            w = pltpu.make_async_copy(
                E_ref.at[pl.ds(0, 1)], gbuf.at[pl.ds(0, 1)], g_sems.at[e]
            )

            def wfn(i, c, w=w):
                w.wait()
                return c

            lax.fori_loop(0, ER, wfn, 0, unroll=8)
            rows = pl.ds(e * ER, ER)
            xbuf[rows, :] = (
                gbuf[rows, :] * valid_ref[rows, :]
            ).astype(jnp.bfloat16)

        pb = (1 - p) * 2 * ER
        p1 = {}
        for L in L_ORDER:
            p1[L] = pltpu.make_async_remote_copy(
                src_ref=xbuf.at[pl.ds(pb + L * ER, ER)],
                dst_ref=p1_buf.at[pl.ds(L * ER, ER)],
                send_sem=p1_send.at[L],
                recv_sem=p1_recv.at[L],
                device_id=partner_dev,
                device_id_type=pl.DeviceIdType.MESH,
            )
            p1[L].start()

        ring = {}
        mirrors = []
        out_dmas = []

        def ship(rows, sem):
            out_stage[rows, :] = comm[rows, :].astype(jnp.float32)
            d = pltpu.make_async_copy(
                out_stage.at[rows], hbm_out.at[rows], sem
            )
            d.start()
            out_dmas.append(d)

        def start_hop(L, h):
            o = (pos - h) % N_RING if L < 3 else (pos + h) % N_RING
            rows = pl.ds(o * STRIPE + mb + L * ER, ER)
            d = pltpu.make_async_remote_copy(
                src_ref=comm.at[rows],
                dst_ref=comm.at[rows],
                send_sem=ring_send.at[L, h],
                recv_sem=ring_recv.at[L, h],
                device_id=right_dev if L < 3 else left_dev,
                device_id_type=pl.DeviceIdType.MESH,
            )
            d.start()
            ring[(L, h)] = d

        def mirror_send(s, h):
            o = jnp.where(p == 0, (pos - h) % N_RING, (pos + h) % N_RING)
            rows = pl.ds(o * STRIPE + p * 6 * ER, 2 * ER)
            d = pltpu.make_async_remote_copy(
                src_ref=comm.at[rows],
                dst_ref=comm.at[rows],
                send_sem=mir_send.at[s],
                recv_sem=mir_recv.at[s],
                device_id=partner_dev,
                device_id_type=pl.DeviceIdType.MESH,
            )
            d.start()
            mirrors.append(d)

        def mirror_wait_ship(s):
            o = jnp.where(p == 0, (pos + s) % N_RING, (pos - s) % N_RING)
            rows = pl.ds(o * STRIPE + (1 - p) * 6 * ER, 2 * ER)
            d = pltpu.make_async_remote_copy(
                src_ref=comm.at[rows],
                dst_ref=comm.at[rows],
                send_sem=mir_send.at[s],
                recv_sem=mir_recv.at[s],
                device_id=partner_dev,
                device_id_type=pl.DeviceIdType.MESH,
            )
            d.wait_recv()
            ship(rows, out_mir_sems.at[s])

        for L in L_ORDER:
            p1[L].wait_recv()
            rows = pl.ds(pos * STRIPE + mb + L * ER, ER)
            comm[rows, :] = (
                xbuf[pl.ds(mb + L * ER, ER), :]
                + p1_buf[pl.ds(L * ER, ER), :]
            )
            start_hop(L, 0)
        mirror_send(0, 0)
        ship(pl.ds(pos * STRIPE + mb, 6 * ER), out_own_sem)

        for h in range(1, N_HOPS + 1):
            last = h == N_HOPS
            for L in L_ORDER:
                ring[(L, h - 1)].wait_recv()
                if not last:
                    start_hop(L, h)
            mirror_send(h, h)
            ship(
                pl.ds(((pos - h) % N_RING) * STRIPE + mb, 3 * ER),
                out_ring_sems.at[0, h - 1],
            )
            ship(
                pl.ds(((pos + h) % N_RING) * STRIPE + mb + 3 * ER, 3 * ER),
                out_ring_sems.at[1, h - 1],
            )
            mirror_wait_ship(h - 1)
        mirror_wait_ship(N_HOPS)

        for L in L_ORDER:
            p1[L].wait_send()
        for d in ring.values():
            d.wait_send()
        for d in mirrors:
            d.wait_send()
        for d in out_dmas:
            d.wait()

    my_x = lax.axis_index("x")
    my_y = lax.axis_index("y")
    my_z = lax.axis_index("z")
    pos = _ring_pos(my_y, my_z)
    ids_stripe = lax.dynamic_slice(ids, (pos * STRIPE,), (STRIPE,))
    local = ids_stripe - my_x * V_LOCAL
    valid = (local >= 0) & (local < V_LOCAL)
    clip_ids = jnp.clip(local, 0, V_LOCAL - 1).astype(jnp.int32)
    validf = valid.astype(jnp.float32)[:, None]

    return pl.pallas_call(
        body,
        out_shape=jax.ShapeDtypeStruct((T, D), jnp.float32),
        in_specs=[
            pl.BlockSpec(memory_space=pl.ANY),
            pl.BlockSpec(memory_space=pltpu.SMEM),
            pl.BlockSpec(memory_space=pltpu.VMEM),
        ],
        out_specs=pl.BlockSpec(memory_space=pl.ANY),
        scratch_shapes=[
            pltpu.VMEM((T, D), jnp.bfloat16),
            pltpu.VMEM((T, D), jnp.float32),
            pltpu.VMEM((6 * ER, D), jnp.bfloat16),
            pltpu.VMEM((STRIPE, D), jnp.float32),
            pltpu.VMEM((STRIPE, D), jnp.bfloat16),
            pltpu.SemaphoreType.DMA((8,)),
            pltpu.SemaphoreType.DMA((6,)),
            pltpu.SemaphoreType.DMA((6,)),
            pltpu.SemaphoreType.DMA((6, N_HOPS)),
            pltpu.SemaphoreType.DMA((6, N_HOPS)),
            pltpu.SemaphoreType.DMA((N_RING,)),
            pltpu.SemaphoreType.DMA((N_RING,)),
            pltpu.SemaphoreType.DMA((2, N_HOPS)),
            pltpu.SemaphoreType.DMA((N_RING,)),
            pltpu.SemaphoreType.DMA,
        ],
        compiler_params=pltpu.CompilerParams(
            collective_id=0,
            vmem_limit_bytes=62 * 1024 * 1024,
        ),
    )(E, clip_ids, validf)


# device time: 123237 ns/iter; 2.2795x vs baseline; 1.0733x over previous
import jax
import jax.numpy as jnp
from jax import lax
from jax.experimental import pallas as pl
from jax.experimental.pallas import tpu as pltpu

N_RING = 8
N_HOPS = N_RING - 1
T = 4096
D = 2048
V_LOCAL = 8192
STRIPE = T // N_RING
ER = STRIPE // 8
L_ORDER = (0, 3, 1, 4, 2, 5)


def _ring_pos(y, z):
    return jnp.where(y == 0, z, 7 - z)


def _ring_coords(p):
    y = jnp.where(p < 4, 0, 1)
    z = jnp.where(p < 4, p, 7 - p)
    return y, z


def kernel(ids, E):
    def body(E_ref, idsc_ref, valid_ref, hbm_out, comm, out_stage, p1_buf,
             gbuf, xbuf, g_sems,
             p1_send, p1_recv, ring_send, ring_recv, mir_send, mir_recv,
             out_ring_sems, out_mir_sems, out_own_sem):
        p = lax.axis_index("x")
        my_y = lax.axis_index("y")
        my_z = lax.axis_index("z")
        pos = _ring_pos(my_y, my_z)
        mb = p * 2 * ER

        r_y, r_z = _ring_coords((pos + 1) % N_RING)
        l_y, l_z = _ring_coords((pos - 1) % N_RING)
        right_dev = (p, r_y, r_z)
        left_dev = (p, l_y, l_z)
        partner_dev = (1 - p, my_y, my_z)

        A_ORDER = (2, 5, 0, 3, 4, 7, 1, 6)
        for e in A_ORDER:
            base = e * ER

            def issue(i, c, base=base, e=e):
                r = base + i
                pltpu.make_async_copy(
                    E_ref.at[pl.ds(idsc_ref[r], 1)],
                    gbuf.at[pl.ds(r, 1)],
                    g_sems.at[e],
                ).start()
                return c

            lax.fori_loop(0, ER, issue, 0, unroll=8)

        barrier = pltpu.get_barrier_semaphore()
        for dev in (partner_dev, right_dev, left_dev):
            pl.semaphore_signal(
                barrier, inc=1, device_id=dev,
                device_id_type=pl.DeviceIdType.MESH,
            )
        pl.semaphore_wait(barrier, 3)

        def p1_descriptor(a):
            Lp = a - (1 - p) * 2
            return pltpu.make_async_remote_copy(
                src_ref=xbuf.at[pl.ds(a * ER, ER)],
                dst_ref=p1_buf.at[pl.ds(Lp * ER, ER)],
                send_sem=p1_send.at[a],
                recv_sem=p1_recv.at[Lp],
                device_id=partner_dev,
                device_id_type=pl.DeviceIdType.MESH,
            )

        for a in A_ORDER:
            w = pltpu.make_async_copy(
                E_ref.at[pl.ds(0, 1)], gbuf.at[pl.ds(0, 1)], g_sems.at[a]
            )

            def wfn(i, c, w=w):
                w.wait()
                return c

            lax.fori_loop(0, ER, wfn, 0, unroll=8)
            rows = pl.ds(a * ER, ER)
            xbuf[rows, :] = (
                gbuf[rows, :] * valid_ref[rows, :]
            ).astype(jnp.bfloat16)
            Lp = a - (1 - p) * 2

            @pl.when((Lp >= 0) & (Lp <= 5))
            def _(a=a):
                p1_descriptor(a).start()

        ring = {}
        mirrors = []
        out_dmas = []

        def ship(rows, sem):
            out_stage[rows, :] = comm[rows, :].astype(jnp.float32)
            d = pltpu.make_async_copy(
                out_stage.at[rows], hbm_out.at[rows], sem
            )
            d.start()
            out_dmas.append(d)

        def start_hop(L, h):
            o = (pos - h) % N_RING if L < 3 else (pos + h) % N_RING
            rows = pl.ds(o * STRIPE + mb + L * ER, ER)
            d = pltpu.make_async_remote_copy(
                src_ref=comm.at[rows],
                dst_ref=comm.at[rows],
                send_sem=ring_send.at[L, h],
                recv_sem=ring_recv.at[L, h],
                device_id=right_dev if L < 3 else left_dev,
                device_id_type=pl.DeviceIdType.MESH,
            )
            d.start()
            ring[(L, h)] = d

        def mirror_send(s, h):
            o = jnp.where(p == 0, (pos - h) % N_RING, (pos + h) % N_RING)
            rows = pl.ds(o * STRIPE + p * 6 * ER, 2 * ER)
            d = pltpu.make_async_remote_copy(
                src_ref=comm.at[rows],
                dst_ref=comm.at[rows],
                send_sem=mir_send.at[s],
                recv_sem=mir_recv.at[s],
                device_id=partner_dev,
                device_id_type=pl.DeviceIdType.MESH,
            )
            d.start()
            mirrors.append(d)

        def mirror_wait_ship(s):
            o = jnp.where(p == 0, (pos + s) % N_RING, (pos - s) % N_RING)
            rows = pl.ds(o * STRIPE + (1 - p) * 6 * ER, 2 * ER)
            d = pltpu.make_async_remote_copy(
                src_ref=comm.at[rows],
                dst_ref=comm.at[rows],
                send_sem=mir_send.at[s],
                recv_sem=mir_recv.at[s],
                device_id=partner_dev,
                device_id_type=pl.DeviceIdType.MESH,
            )
            d.wait_recv()
            ship(rows, out_mir_sems.at[s])

        for L in L_ORDER:
            pltpu.make_async_remote_copy(
                src_ref=p1_buf.at[pl.ds(L * ER, ER)],
                dst_ref=p1_buf.at[pl.ds(L * ER, ER)],
                send_sem=p1_send.at[L],
                recv_sem=p1_recv.at[L],
                device_id=partner_dev,
                device_id_type=pl.DeviceIdType.MESH,
            ).wait_recv()
            rows = pl.ds(pos * STRIPE + mb + L * ER, ER)
            comm[rows, :] = (
                xbuf[pl.ds(mb + L * ER, ER), :]
                + p1_buf[pl.ds(L * ER, ER), :]
            )
            start_hop(L, 0)
        mirror_send(0, 0)
        ship(pl.ds(pos * STRIPE + mb, 6 * ER), out_own_sem)

        for h in range(1, N_HOPS + 1):
            last = h == N_HOPS
            for L in L_ORDER:
                ring[(L, h - 1)].wait_recv()
                if not last:
                    start_hop(L, h)
            mirror_send(h, h)
            ship(
                pl.ds(((pos - h) % N_RING) * STRIPE + mb, 3 * ER),
                out_ring_sems.at[0, h - 1],
            )
            ship(
                pl.ds(((pos + h) % N_RING) * STRIPE + mb + 3 * ER, 3 * ER),
                out_ring_sems.at[1, h - 1],
            )
            mirror_wait_ship(h - 1)
        mirror_wait_ship(N_HOPS)

        for a in A_ORDER:
            Lp = a - (1 - p) * 2

            @pl.when((Lp >= 0) & (Lp <= 5))
            def _(a=a):
                p1_descriptor(a).wait_send()
        for d in ring.values():
            d.wait_send()
        for d in mirrors:
            d.wait_send()
        for d in out_dmas:
            d.wait()

    my_x = lax.axis_index("x")
    my_y = lax.axis_index("y")
    my_z = lax.axis_index("z")
    pos = _ring_pos(my_y, my_z)
    ids_stripe = lax.dynamic_slice(ids, (pos * STRIPE,), (STRIPE,))
    local = ids_stripe - my_x * V_LOCAL
    valid = (local >= 0) & (local < V_LOCAL)
    clip_ids = jnp.clip(local, 0, V_LOCAL - 1).astype(jnp.int32)
    validf = valid.astype(jnp.float32)[:, None]

    return pl.pallas_call(
        body,
        out_shape=jax.ShapeDtypeStruct((T, D), jnp.float32),
        in_specs=[
            pl.BlockSpec(memory_space=pl.ANY),
            pl.BlockSpec(memory_space=pltpu.SMEM),
            pl.BlockSpec(memory_space=pltpu.VMEM),
        ],
        out_specs=pl.BlockSpec(memory_space=pl.ANY),
        scratch_shapes=[
            pltpu.VMEM((T, D), jnp.bfloat16),
            pltpu.VMEM((T, D), jnp.float32),
            pltpu.VMEM((6 * ER, D), jnp.bfloat16),
            pltpu.VMEM((STRIPE, D), jnp.float32),
            pltpu.VMEM((STRIPE, D), jnp.bfloat16),
            pltpu.SemaphoreType.DMA((8,)),
            pltpu.SemaphoreType.DMA((8,)),
            pltpu.SemaphoreType.DMA((6,)),
            pltpu.SemaphoreType.DMA((6, N_HOPS)),
            pltpu.SemaphoreType.DMA((6, N_HOPS)),
            pltpu.SemaphoreType.DMA((N_RING,)),
            pltpu.SemaphoreType.DMA((N_RING,)),
            pltpu.SemaphoreType.DMA((2, N_HOPS)),
            pltpu.SemaphoreType.DMA((N_RING,)),
            pltpu.SemaphoreType.DMA,
        ],
        compiler_params=pltpu.CompilerParams(
            collective_id=0,
            vmem_limit_bytes=62 * 1024 * 1024,
        ),
    )(E, clip_ids, validf)
